# Initial kernel scaffold; baseline (speedup 1.0000x reference)
import jax
import jax.numpy as jnp
from jax.experimental import pallas as pl


def kernel(coef, temb_rev_p, w_lldc, w9p, bias, bw, lowlight, data_concate, brightness, y_init, noise_rev):
    raise NotImplementedError("write your pallas kernel here")


# speedup vs baseline: 2.5057x; 2.5057x over previous
import jax
import jax.numpy as jnp
from jax import lax
from jax.experimental import pallas as pl
from jax.experimental.pallas import tpu as pltpu


def _make_body(H, W, R, KP, T, C):
    HW = H * W
    TCOL = 10 * R

    taps_meta = []
    for kh in range(3):
        for kw in range(3):
            dh, dw = kh - 1, kw - 1
            delta = dh * W + dw
            taps_meta.append(((-delta) % HW, dh, dw))

    def body(coef_s, ct_s, y0_r, epsb_r, nz_r, a0_r, out_r,
             y_s, taps_s, mask_s):
        t = pl.program_id(1)

        @pl.when(t == 0)
        def _init():
            y_s[...] = y0_r[...]
            taps_s[9 * R:10 * R, :] = epsb_r[...]
            row = lax.broadcasted_iota(jnp.int32, (KP - TCOL, HW), 0)
            taps_s[TCOL:KP, :] = jnp.where(row == 0, 1.0, 0.0)
            hw_idx = lax.broadcasted_iota(jnp.int32, (1, HW), 1)
            h_pos = hw_idx // W
            w_pos = hw_idx % W
            for k, (_, dh, dw) in enumerate(taps_meta):
                valid = jnp.ones((1, HW), jnp.bool_)
                if dh == -1:
                    valid = jnp.logical_and(valid, h_pos >= 1)
                elif dh == 1:
                    valid = jnp.logical_and(valid, h_pos <= H - 2)
                if dw == -1:
                    valid = jnp.logical_and(valid, w_pos >= 1)
                elif dw == 1:
                    valid = jnp.logical_and(valid, w_pos <= W - 2)
                mask_s[k:k + 1, :] = valid.astype(jnp.float32)

        y = y_s[...]
        for k, (shift, dh, dw) in enumerate(taps_meta):
            tap = y if shift == 0 else pltpu.roll(y, shift=shift, axis=1)
            if not (dh == 0 and dw == 0):
                tap = tap * mask_s[k:k + 1, :]
            taps_s[k * R:(k + 1) * R, :] = tap

        c1 = coef_s[0, t]
        c2 = coef_s[1, t]
        sv = coef_s[2, t]
        A = a0_r[...] * c2
        rc = lax.broadcasted_iota(jnp.int32, (R, 1), 0) % C
        tv = jnp.where(rc == 0, ct_s[0, t],
                       jnp.where(rc == 1, ct_s[1, t], ct_s[2, t]))
        lane = lax.broadcasted_iota(jnp.int32, (R, KP), 1)
        A = jnp.where(lane == TCOL, tv, A)

        eps = jnp.dot(A, taps_s[...], preferred_element_type=jnp.float32)

        nz = nz_r[...].reshape(R, HW)
        y_new = c1 * y - eps + sv * nz

        @pl.when(t == T - 1)
        def _finish():
            out_r[...] = jnp.clip(y_new, -1.0, 1.0)

        y_s[...] = y_new

    return body


def kernel(coef, temb_rev_p, w_lldc, w9p, bias, bw,
           lowlight, data_concate, brightness, y_init, noise_rev):
    B, C, H, W = lowlight.shape
    T = noise_rev.shape[1]
    HW = H * W
    Cp = w9p.shape[0]
    NC = 2
    Bh = B // NC
    R = Bh * C
    KP = ((10 * R + 1 + 7) // 8) * 8

    x2 = jnp.concatenate([lowlight.astype(jnp.float32),
                          data_concate.astype(jnp.float32)], axis=1)
    eps_base = lax.conv_general_dilated(
        x2, w_lldc, (1, 1), 'SAME',
        dimension_numbers=('NCHW', 'HWIO', 'NCHW'))
    eps_base = (eps_base + bias[None, :, None, None]
                + brightness.astype(jnp.float32)[:, None, None, None]
                * bw[None, :, None, None])

    w_small = w9p.reshape(Cp, 9, Cp)[:C, :, :C]
    eyeB = jnp.eye(Bh, dtype=jnp.float32)
    W_big = jnp.concatenate(
        [jnp.kron(eyeB, w_small[:, k, :]) for k in range(9)], axis=1)
    A0 = (jnp.zeros((R, KP), jnp.float32)
          .at[:, :9 * R].set(W_big)
          .at[:, 9 * R:10 * R].set(jnp.eye(R, dtype=jnp.float32)))
    ct = (coef[1][:, None] * temb_rev_p[:, :C, 0]).T

    y0 = y_init.astype(jnp.float32).reshape(NC, R, HW)
    epsb = eps_base.reshape(NC, R, HW)
    noise = noise_rev.astype(jnp.float32).reshape(B, T, C, HW)

    out = pl.pallas_call(
        _make_body(H, W, R, KP, T, C),
        out_shape=jax.ShapeDtypeStruct((NC, R, HW), jnp.float32),
        grid=(NC, T),
        in_specs=[
            pl.BlockSpec(memory_space=pltpu.MemorySpace.SMEM),
            pl.BlockSpec(memory_space=pltpu.MemorySpace.SMEM),
            pl.BlockSpec((None, R, HW), lambda c, t: (c, 0, 0)),
            pl.BlockSpec((None, R, HW), lambda c, t: (c, 0, 0)),
            pl.BlockSpec((Bh, None, C, HW), lambda c, t: (c, t, 0, 0)),
            pl.BlockSpec((R, KP), lambda c, t: (0, 0)),
        ],
        out_specs=pl.BlockSpec((None, R, HW), lambda c, t: (c, 0, 0)),
        scratch_shapes=[
            pltpu.VMEM((R, HW), jnp.float32),
            pltpu.VMEM((KP, HW), jnp.float32),
            pltpu.VMEM((9, HW), jnp.float32),
        ],
        compiler_params=pltpu.CompilerParams(
            dimension_semantics=("parallel", "arbitrary")),
    )(coef, ct, y0, epsb, noise, A0)

    return out.reshape(B, C, H, W)


# speedup vs baseline: 3.8209x; 1.5248x over previous
import jax
import jax.numpy as jnp
from jax import lax
from jax.experimental import pallas as pl
from jax.experimental.pallas import tpu as pltpu


def _make_body(H, W, R, KP, T, C):
    HW = H * W
    TCOL = 10 * R

    taps_meta = []
    for kh in range(3):
        for kw in range(3):
            dh, dw = kh - 1, kw - 1
            delta = dh * W + dw
            taps_meta.append(((-delta) % HW, dh, dw))

    def body(coef_s, ct_s, y0_r, epsb_r, nz_r, a0_r, out_r,
             y_s, taps_s, mask_s):
        t = pl.program_id(1)

        @pl.when(t == 0)
        def _init():
            y_s[...] = y0_r[...]
            taps_s[9 * R:10 * R, :] = epsb_r[...]
            row = lax.broadcasted_iota(jnp.int32, (KP - TCOL, HW), 0)
            taps_s[TCOL:KP, :] = jnp.where(row == 0, 1.0, 0.0)
            hw_idx = lax.broadcasted_iota(jnp.int32, (1, HW), 1)
            h_pos = hw_idx // W
            w_pos = hw_idx % W
            for k, (_, dh, dw) in enumerate(taps_meta):
                valid = jnp.ones((1, HW), jnp.bool_)
                if dh == -1:
                    valid = jnp.logical_and(valid, h_pos >= 1)
                elif dh == 1:
                    valid = jnp.logical_and(valid, h_pos <= H - 2)
                if dw == -1:
                    valid = jnp.logical_and(valid, w_pos >= 1)
                elif dw == 1:
                    valid = jnp.logical_and(valid, w_pos <= W - 2)
                mask_s[k:k + 1, :] = valid.astype(jnp.float32)

        y = y_s[...]
        for k, (shift, dh, dw) in enumerate(taps_meta):
            tap = y if shift == 0 else pltpu.roll(y, shift=shift, axis=1)
            if not (dh == 0 and dw == 0):
                tap = tap * mask_s[k:k + 1, :]
            taps_s[k * R:(k + 1) * R, :] = tap

        c1 = coef_s[0, t]
        c2 = coef_s[1, t]
        sv = coef_s[2, t]
        A = a0_r[...] * c2
        rc = lax.broadcasted_iota(jnp.int32, (R, 1), 0) % C
        tv = jnp.where(rc == 0, ct_s[0, t],
                       jnp.where(rc == 1, ct_s[1, t], ct_s[2, t]))
        lane = lax.broadcasted_iota(jnp.int32, (R, KP), 1)
        A = jnp.where(lane == TCOL, tv, A)

        eps = jnp.dot(A, taps_s[...], preferred_element_type=jnp.float32)

        nz = nz_r[...].reshape(R, HW)
        y_new = c1 * y - eps + sv * nz

        @pl.when(t == T - 1)
        def _finish():
            out_r[...] = jnp.clip(y_new, -1.0, 1.0)

        y_s[...] = y_new

    return body


def kernel(coef, temb_rev_p, w_lldc, w9p, bias, bw,
           lowlight, data_concate, brightness, y_init, noise_rev):
    B, C, H, W = lowlight.shape
    T = noise_rev.shape[1]
    HW = H * W
    Cp = w9p.shape[0]
    NC = 2
    Bh = B // NC
    R = Bh * C
    KP = ((10 * R + 1 + 7) // 8) * 8

    x2 = jnp.concatenate([lowlight.astype(jnp.float32),
                          data_concate.astype(jnp.float32)], axis=1)
    eps_base = lax.conv_general_dilated(
        x2, w_lldc, (1, 1), 'SAME',
        dimension_numbers=('NCHW', 'HWIO', 'NCHW'))
    eps_base = (eps_base + bias[None, :, None, None]
                + brightness.astype(jnp.float32)[:, None, None, None]
                * bw[None, :, None, None])

    w_small = w9p.reshape(Cp, 9, Cp)[:C, :, :C]
    eyeB = jnp.eye(Bh, dtype=jnp.float32)
    W_big = jnp.concatenate(
        [jnp.kron(eyeB, w_small[:, k, :]) for k in range(9)], axis=1)
    A0 = (jnp.zeros((R, KP), jnp.float32)
          .at[:, :9 * R].set(W_big)
          .at[:, 9 * R:10 * R].set(jnp.eye(R, dtype=jnp.float32)))
    ct = (coef[1][:, None] * temb_rev_p[:, :C, 0]).T

    y0 = y_init.astype(jnp.float32).reshape(NC, R, HW)
    epsb = eps_base.reshape(NC, R, HW)
    noise = noise_rev.astype(jnp.float32)

    out = pl.pallas_call(
        _make_body(H, W, R, KP, T, C),
        out_shape=jax.ShapeDtypeStruct((NC, R, HW), jnp.float32),
        grid=(NC, T),
        in_specs=[
            pl.BlockSpec(memory_space=pltpu.MemorySpace.SMEM),
            pl.BlockSpec(memory_space=pltpu.MemorySpace.SMEM),
            pl.BlockSpec((None, R, HW), lambda c, t: (c, 0, 0)),
            pl.BlockSpec((None, R, HW), lambda c, t: (c, 0, 0)),
            pl.BlockSpec((Bh, None, C, H, W), lambda c, t: (c, t, 0, 0, 0)),
            pl.BlockSpec((R, KP), lambda c, t: (0, 0)),
        ],
        out_specs=pl.BlockSpec((None, R, HW), lambda c, t: (c, 0, 0)),
        scratch_shapes=[
            pltpu.VMEM((R, HW), jnp.float32),
            pltpu.VMEM((KP, HW), jnp.float32),
            pltpu.VMEM((9, HW), jnp.float32),
        ],
        compiler_params=pltpu.CompilerParams(
            dimension_semantics=("parallel", "arbitrary")),
    )(coef, ct, y0, epsb, noise, A0)

    return out.reshape(B, C, H, W)


# speedup vs baseline: 4.3523x; 1.1391x over previous
import jax
import jax.numpy as jnp
from jax import lax
from jax.experimental import pallas as pl
from jax.experimental.pallas import tpu as pltpu


def _make_body(H, W, Bh, R, KP, T, C):
    HW = H * W

    taps_meta = []
    for kh in range(3):
        for kw in range(3):
            dh, dw = kh - 1, kw - 1
            delta = dh * W + dw
            taps_meta.append(((-delta) % HW, dh, dw))

    def body(coef_s, ct_s, addv_r, ll_r, dc_r, y5_r, nz_r,
             a0_r, all_r, adc_r, out_r, y_s, taps_s, mask_s):
        t = pl.program_id(1)

        def build_taps(src):
            for k, (shift, dh, dw) in enumerate(taps_meta):
                tap = src if shift == 0 else pltpu.roll(src, shift=shift,
                                                       axis=1)
                if not (dh == 0 and dw == 0):
                    tap = tap * mask_s[k:k + 1, :]
                taps_s[k * R:(k + 1) * R, :] = tap

        @pl.when(t == 0)
        def _init():
            hw_idx = lax.broadcasted_iota(jnp.int32, (1, HW), 1)
            h_pos = hw_idx // W
            w_pos = hw_idx % W
            for k, (_, dh, dw) in enumerate(taps_meta):
                valid = jnp.ones((1, HW), jnp.bool_)
                if dh == -1:
                    valid = jnp.logical_and(valid, h_pos >= 1)
                elif dh == 1:
                    valid = jnp.logical_and(valid, h_pos <= H - 2)
                if dw == -1:
                    valid = jnp.logical_and(valid, w_pos >= 1)
                elif dw == 1:
                    valid = jnp.logical_and(valid, w_pos <= W - 2)
                mask_s[k:k + 1, :] = valid.astype(jnp.float32)

            build_taps(ll_r[...].reshape(R, HW))
            e1 = jnp.dot(all_r[...], taps_s[0:9 * R, :],
                         preferred_element_type=jnp.float32)
            build_taps(dc_r[...].reshape(R, HW))
            e2 = jnp.dot(adc_r[...], taps_s[0:9 * R, :],
                         preferred_element_type=jnp.float32)
            taps_s[9 * R:10 * R, :] = e1 + e2 + addv_r[:, :1]
            y_s[...] = y5_r[...].reshape(R, HW)

        y = y_s[...]
        build_taps(y)
        eps0 = jnp.dot(a0_r[...], taps_s[...],
                       preferred_element_type=jnp.float32)

        c1 = coef_s[0, t]
        c2 = coef_s[1, t]
        sv = coef_s[2, t]
        rc = lax.broadcasted_iota(jnp.int32, (R, 1), 0) % C
        tv = jnp.where(rc == 0, ct_s[0, t],
                       jnp.where(rc == 1, ct_s[1, t], ct_s[2, t]))
        nz = nz_r[...].reshape(R, HW)
        y_new = c1 * y - c2 * eps0 - tv + sv * nz

        @pl.when(t == T - 1)
        def _finish():
            out_r[...] = jnp.clip(y_new, -1.0, 1.0).reshape(Bh, C, H, W)

        y_s[...] = y_new

    return body


def _kron_taps(w_ock, Bh):
    eyeB = jnp.eye(Bh, dtype=jnp.float32)
    return jnp.concatenate(
        [jnp.kron(eyeB, w_ock[:, k, :]) for k in range(9)], axis=1)


def kernel(coef, temb_rev_p, w_lldc, w9p, bias, bw,
           lowlight, data_concate, brightness, y_init, noise_rev):
    B, C, H, W = lowlight.shape
    T = noise_rev.shape[1]
    HW = H * W
    Cp = w9p.shape[0]
    NC = 2
    Bh = B // NC
    R = Bh * C
    KP = 10 * R

    w_y = w9p.reshape(Cp, 9, Cp)[:C, :, :C]
    A0 = jnp.concatenate(
        [_kron_taps(w_y, Bh), jnp.eye(R, dtype=jnp.float32)], axis=1)
    w_c = jnp.transpose(w_lldc, (3, 0, 1, 2)).reshape(C, 9, 2 * C)
    A_ll = _kron_taps(w_c[:, :, :C], Bh)
    A_dc = _kron_taps(w_c[:, :, C:], Bh)
    ct = (coef[1][:, None] * temb_rev_p[:, :C, 0]).T
    addv = (bias[None, :] + brightness.astype(jnp.float32)[:, None]
            * bw[None, :]).reshape(NC, R, 1)
    addv = addv + jnp.zeros((NC, R, 128), jnp.float32)

    out = pl.pallas_call(
        _make_body(H, W, Bh, R, KP, T, C),
        out_shape=jax.ShapeDtypeStruct((B, C, H, W), jnp.float32),
        grid=(NC, T),
        in_specs=[
            pl.BlockSpec(memory_space=pltpu.MemorySpace.SMEM),
            pl.BlockSpec(memory_space=pltpu.MemorySpace.SMEM),
            pl.BlockSpec((None, R, 128), lambda c, t: (c, 0, 0)),
            pl.BlockSpec((Bh, C, H, W), lambda c, t: (c, 0, 0, 0)),
            pl.BlockSpec((Bh, C, H, W), lambda c, t: (c, 0, 0, 0)),
            pl.BlockSpec((Bh, C, H, W), lambda c, t: (c, 0, 0, 0)),
            pl.BlockSpec((Bh, None, C, H, W),
                         lambda c, t: (c, t, 0, 0, 0)),
            pl.BlockSpec((R, KP), lambda c, t: (0, 0)),
            pl.BlockSpec((R, 9 * R), lambda c, t: (0, 0)),
            pl.BlockSpec((R, 9 * R), lambda c, t: (0, 0)),
        ],
        out_specs=pl.BlockSpec((Bh, C, H, W), lambda c, t: (c, 0, 0, 0)),
        scratch_shapes=[
            pltpu.VMEM((R, HW), jnp.float32),
            pltpu.VMEM((KP, HW), jnp.float32),
            pltpu.VMEM((9, HW), jnp.float32),
        ],
        compiler_params=pltpu.CompilerParams(
            dimension_semantics=("parallel", "arbitrary")),
    )(coef, ct, addv, lowlight.astype(jnp.float32),
      data_concate.astype(jnp.float32), y_init.astype(jnp.float32),
      noise_rev.astype(jnp.float32), A0, A_ll, A_dc)

    return out


# speedup vs baseline: 4.4971x; 1.0333x over previous
import jax
import jax.numpy as jnp
from jax import lax
from jax.experimental import pallas as pl
from jax.experimental.pallas import tpu as pltpu


def _make_body(H, W, Bh, R, KP, T, C):
    HW = H * W

    taps_meta = []
    for kh in range(3):
        for kw in range(3):
            dh, dw = kh - 1, kw - 1
            delta = dh * W + dw
            taps_meta.append(((-delta) % HW, dh, dw))

    def body(coef_s, ct_s, addv_r, ll_r, dc_r, y5_r, nz_r,
             a0_r, all_r, adc_r, out_r, taps_s, mask_s):
        t = pl.program_id(1)

        def build_taps(src, store_center):
            for k, (shift, dh, dw) in enumerate(taps_meta):
                if dh == 0 and dw == 0:
                    if store_center:
                        taps_s[k * R:(k + 1) * R, :] = src
                    continue
                tap = pltpu.roll(src, shift=shift, axis=1)
                tap = tap * mask_s[k:k + 1, :]
                taps_s[k * R:(k + 1) * R, :] = tap

        @pl.when(t == 0)
        def _init():
            hw_idx = lax.broadcasted_iota(jnp.int32, (1, HW), 1)
            h_pos = hw_idx // W
            w_pos = hw_idx % W
            for k, (_, dh, dw) in enumerate(taps_meta):
                valid = jnp.ones((1, HW), jnp.bool_)
                if dh == -1:
                    valid = jnp.logical_and(valid, h_pos >= 1)
                elif dh == 1:
                    valid = jnp.logical_and(valid, h_pos <= H - 2)
                if dw == -1:
                    valid = jnp.logical_and(valid, w_pos >= 1)
                elif dw == 1:
                    valid = jnp.logical_and(valid, w_pos <= W - 2)
                mask_s[k:k + 1, :] = valid.astype(jnp.float32)

            build_taps(ll_r[...].reshape(R, HW), store_center=True)
            e1 = jnp.dot(all_r[...], taps_s[0:9 * R, :],
                         preferred_element_type=jnp.float32)
            build_taps(dc_r[...].reshape(R, HW), store_center=True)
            e2 = jnp.dot(adc_r[...], taps_s[0:9 * R, :],
                         preferred_element_type=jnp.float32)
            taps_s[9 * R:10 * R, :] = e1 + e2 + addv_r[:, :1]
            taps_s[4 * R:5 * R, :] = y5_r[...].reshape(R, HW)

        y = taps_s[4 * R:5 * R, :]
        build_taps(y, store_center=False)
        eps0 = jnp.dot(a0_r[...], taps_s[...],
                       preferred_element_type=jnp.float32)

        c1 = coef_s[0, t]
        c2 = coef_s[1, t]
        sv = coef_s[2, t]
        rc = lax.broadcasted_iota(jnp.int32, (R, 1), 0) % C
        tv = jnp.where(rc == 0, ct_s[0, t],
                       jnp.where(rc == 1, ct_s[1, t], ct_s[2, t]))
        nz = nz_r[...].reshape(R, HW)
        y_new = c1 * y - c2 * eps0 - tv + sv * nz

        @pl.when(t == T - 1)
        def _finish():
            out_r[...] = jnp.clip(y_new, -1.0, 1.0).reshape(Bh, C, H, W)

        taps_s[4 * R:5 * R, :] = y_new

    return body


def _kron_taps(w_ock, Bh):
    C_out, _, C_in = w_ock.shape
    eyeB = jnp.eye(Bh, dtype=jnp.float32)
    big = (eyeB[:, None, None, :, None]
           * w_ock[None, :, :, None, :])
    return big.reshape(Bh * C_out, 9 * Bh * C_in)


def kernel(coef, temb_rev_p, w_lldc, w9p, bias, bw,
           lowlight, data_concate, brightness, y_init, noise_rev):
    B, C, H, W = lowlight.shape
    T = noise_rev.shape[1]
    HW = H * W
    Cp = w9p.shape[0]
    NC = 2
    Bh = B // NC
    R = Bh * C
    KP = 10 * R

    w_y = w9p.reshape(Cp, 9, Cp)[:C, :, :C]
    A0 = jnp.concatenate(
        [_kron_taps(w_y, Bh), jnp.eye(R, dtype=jnp.float32)], axis=1)
    w_c = jnp.transpose(w_lldc, (3, 0, 1, 2)).reshape(C, 9, 2 * C)
    A_ll = _kron_taps(w_c[:, :, :C], Bh)
    A_dc = _kron_taps(w_c[:, :, C:], Bh)
    ct = (coef[1][:, None] * temb_rev_p[:, :C, 0]).T
    addv = (bias[None, :] + brightness.astype(jnp.float32)[:, None]
            * bw[None, :]).reshape(NC, R, 1)
    addv = addv + jnp.zeros((NC, R, 128), jnp.float32)

    out = pl.pallas_call(
        _make_body(H, W, Bh, R, KP, T, C),
        out_shape=jax.ShapeDtypeStruct((B, C, H, W), jnp.float32),
        grid=(NC, T),
        in_specs=[
            pl.BlockSpec(memory_space=pltpu.MemorySpace.SMEM),
            pl.BlockSpec(memory_space=pltpu.MemorySpace.SMEM),
            pl.BlockSpec((None, R, 128), lambda c, t: (c, 0, 0)),
            pl.BlockSpec((Bh, C, H, W), lambda c, t: (c, 0, 0, 0)),
            pl.BlockSpec((Bh, C, H, W), lambda c, t: (c, 0, 0, 0)),
            pl.BlockSpec((Bh, C, H, W), lambda c, t: (c, 0, 0, 0)),
            pl.BlockSpec((Bh, None, C, H, W),
                         lambda c, t: (c, t, 0, 0, 0)),
            pl.BlockSpec((R, KP), lambda c, t: (0, 0)),
            pl.BlockSpec((R, 9 * R), lambda c, t: (0, 0)),
            pl.BlockSpec((R, 9 * R), lambda c, t: (0, 0)),
        ],
        out_specs=pl.BlockSpec((Bh, C, H, W), lambda c, t: (c, 0, 0, 0)),
        scratch_shapes=[
            pltpu.VMEM((KP, HW), jnp.float32),
            pltpu.VMEM((9, HW), jnp.float32),
        ],
        compiler_params=pltpu.CompilerParams(
            dimension_semantics=("parallel", "arbitrary")),
    )(coef, ct, addv, lowlight.astype(jnp.float32),
      data_concate.astype(jnp.float32), y_init.astype(jnp.float32),
      noise_rev.astype(jnp.float32), A0, A_ll, A_dc)

    return out


# speedup vs baseline: 5.0032x; 1.1125x over previous
import jax
import jax.numpy as jnp
from jax import lax
from jax.experimental import pallas as pl
from jax.experimental.pallas import tpu as pltpu


def _make_body(H, W, Bh, R, KP, T, C, NT):
    HW = H * W
    NJ = T // NT

    taps_meta = []
    for kh in range(3):
        for kw in range(3):
            dh, dw = kh - 1, kw - 1
            delta = dh * W + dw
            taps_meta.append(((-delta) % HW, dh, dw))

    def body(coef_s, ct_s, addv_r, ll_r, dc_r, y5_r, nz_r,
             a0_r, all_r, adc_r, out_r, taps_s, mask_s):
        j = pl.program_id(1)

        def build_taps(src, store_center):
            for k, (shift, dh, dw) in enumerate(taps_meta):
                if dh == 0 and dw == 0:
                    if store_center:
                        taps_s[k * R:(k + 1) * R, :] = src
                    continue
                tap = pltpu.roll(src, shift=shift, axis=1)
                tap = tap * mask_s[k:k + 1, :]
                taps_s[k * R:(k + 1) * R, :] = tap

        @pl.when(j == 0)
        def _init():
            hw_idx = lax.broadcasted_iota(jnp.int32, (1, HW), 1)
            h_pos = hw_idx // W
            w_pos = hw_idx % W
            for k, (_, dh, dw) in enumerate(taps_meta):
                valid = jnp.ones((1, HW), jnp.bool_)
                if dh == -1:
                    valid = jnp.logical_and(valid, h_pos >= 1)
                elif dh == 1:
                    valid = jnp.logical_and(valid, h_pos <= H - 2)
                if dw == -1:
                    valid = jnp.logical_and(valid, w_pos >= 1)
                elif dw == 1:
                    valid = jnp.logical_and(valid, w_pos <= W - 2)
                mask_s[k:k + 1, :] = valid.astype(jnp.float32)

            build_taps(ll_r[...].reshape(R, HW), store_center=True)
            e1 = jnp.dot(all_r[...], taps_s[0:9 * R, :],
                         preferred_element_type=jnp.float32)
            build_taps(dc_r[...].reshape(R, HW), store_center=True)
            e2 = jnp.dot(adc_r[...], taps_s[0:9 * R, :],
                         preferred_element_type=jnp.float32)
            taps_s[9 * R:10 * R, :] = e1 + e2 + addv_r[:, :1]
            taps_s[4 * R:5 * R, :] = y5_r[...].reshape(R, HW)

        rc = lax.broadcasted_iota(jnp.int32, (R, 1), 0) % C
        for jj in range(NT):
            t = j * NT + jj
            y = taps_s[4 * R:5 * R, :]
            build_taps(y, store_center=False)
            eps0 = jnp.dot(a0_r[...], taps_s[...],
                           preferred_element_type=jnp.float32)

            c1 = coef_s[0, t]
            c2 = coef_s[1, t]
            sv = coef_s[2, t]
            tv = jnp.where(rc == 0, ct_s[0, t],
                           jnp.where(rc == 1, ct_s[1, t], ct_s[2, t]))
            nz = nz_r[:, jj].reshape(R, HW)
            y_new = c1 * y - c2 * eps0 - tv + sv * nz

            if jj == NT - 1:
                @pl.when(j == NJ - 1)
                def _finish():
                    out_r[...] = jnp.clip(y_new, -1.0, 1.0).reshape(
                        Bh, C, H, W)

            taps_s[4 * R:5 * R, :] = y_new

    return body


def _kron_taps(w_ock, Bh):
    C_out, _, C_in = w_ock.shape
    eyeB = jnp.eye(Bh, dtype=jnp.float32)
    big = (eyeB[:, None, None, :, None]
           * w_ock[None, :, :, None, :])
    return big.reshape(Bh * C_out, 9 * Bh * C_in)


def kernel(coef, temb_rev_p, w_lldc, w9p, bias, bw,
           lowlight, data_concate, brightness, y_init, noise_rev):
    B, C, H, W = lowlight.shape
    T = noise_rev.shape[1]
    HW = H * W
    Cp = w9p.shape[0]
    NC = 2
    Bh = B // NC
    R = Bh * C
    KP = 10 * R
    NT = 2

    w_y = w9p.reshape(Cp, 9, Cp)[:C, :, :C]
    A0 = jnp.concatenate(
        [_kron_taps(w_y, Bh), jnp.eye(R, dtype=jnp.float32)], axis=1)
    w_c = jnp.transpose(w_lldc, (3, 0, 1, 2)).reshape(C, 9, 2 * C)
    A_ll = _kron_taps(w_c[:, :, :C], Bh)
    A_dc = _kron_taps(w_c[:, :, C:], Bh)
    ct = (coef[1][:, None] * temb_rev_p[:, :C, 0]).T
    addv = (bias[None, :] + brightness.astype(jnp.float32)[:, None]
            * bw[None, :]).reshape(NC, R, 1)
    addv = addv + jnp.zeros((NC, R, 128), jnp.float32)

    out = pl.pallas_call(
        _make_body(H, W, Bh, R, KP, T, C, NT),
        out_shape=jax.ShapeDtypeStruct((B, C, H, W), jnp.float32),
        grid=(NC, T // NT),
        in_specs=[
            pl.BlockSpec(memory_space=pltpu.MemorySpace.SMEM),
            pl.BlockSpec(memory_space=pltpu.MemorySpace.SMEM),
            pl.BlockSpec((None, R, 128), lambda c, t: (c, 0, 0)),
            pl.BlockSpec((Bh, C, H, W), lambda c, t: (c, 0, 0, 0)),
            pl.BlockSpec((Bh, C, H, W), lambda c, t: (c, 0, 0, 0)),
            pl.BlockSpec((Bh, C, H, W), lambda c, t: (c, 0, 0, 0)),
            pl.BlockSpec((Bh, NT, C, H, W),
                         lambda c, t: (c, t, 0, 0, 0)),
            pl.BlockSpec((R, KP), lambda c, t: (0, 0)),
            pl.BlockSpec((R, 9 * R), lambda c, t: (0, 0)),
            pl.BlockSpec((R, 9 * R), lambda c, t: (0, 0)),
        ],
        out_specs=pl.BlockSpec((Bh, C, H, W), lambda c, t: (c, 0, 0, 0)),
        scratch_shapes=[
            pltpu.VMEM((KP, HW), jnp.float32),
            pltpu.VMEM((9, HW), jnp.float32),
        ],
        compiler_params=pltpu.CompilerParams(
            dimension_semantics=("parallel", "arbitrary")),
    )(coef, ct, addv, lowlight.astype(jnp.float32),
      data_concate.astype(jnp.float32), y_init.astype(jnp.float32),
      noise_rev.astype(jnp.float32), A0, A_ll, A_dc)

    return out


# speedup vs baseline: 5.1610x; 1.0315x over previous
import jax
import jax.numpy as jnp
from jax import lax
from jax.experimental import pallas as pl
from jax.experimental.pallas import tpu as pltpu


def _make_body(H, W, Bh, R, KP, T, C, NT):
    HW = H * W
    NJ = T // NT

    taps_meta = []
    for kh in range(3):
        for kw in range(3):
            dh, dw = kh - 1, kw - 1
            delta = dh * W + dw
            taps_meta.append(((-delta) % HW, dh, dw))

    def body(coef_s, ct_s, addv_r, ll_r, dc_r, y5_r, nz_r,
             a0_r, all_r, adc_r, out_r, taps_s, mask_s):
        j = pl.program_id(1)

        def build_taps(src, store_center):
            for k, (shift, dh, dw) in enumerate(taps_meta):
                if dh == 0 and dw == 0:
                    if store_center:
                        taps_s[k * R:(k + 1) * R, :] = src
                    continue
                tap = pltpu.roll(src, shift=shift, axis=1)
                tap = tap * mask_s[k:k + 1, :]
                taps_s[k * R:(k + 1) * R, :] = tap

        @pl.when(j == 0)
        def _init():
            hw_idx = lax.broadcasted_iota(jnp.int32, (1, HW), 1)
            h_pos = hw_idx // W
            w_pos = hw_idx % W
            for k, (_, dh, dw) in enumerate(taps_meta):
                valid = jnp.ones((1, HW), jnp.bool_)
                if dh == -1:
                    valid = jnp.logical_and(valid, h_pos >= 1)
                elif dh == 1:
                    valid = jnp.logical_and(valid, h_pos <= H - 2)
                if dw == -1:
                    valid = jnp.logical_and(valid, w_pos >= 1)
                elif dw == 1:
                    valid = jnp.logical_and(valid, w_pos <= W - 2)
                mask_s[k:k + 1, :] = valid.astype(jnp.float32)

            build_taps(ll_r[...].reshape(R, HW), store_center=True)
            e1 = jnp.dot(all_r[...], taps_s[0:9 * R, :],
                         preferred_element_type=jnp.float32)
            build_taps(dc_r[...].reshape(R, HW), store_center=True)
            e2 = jnp.dot(adc_r[...], taps_s[0:9 * R, :],
                         preferred_element_type=jnp.float32)
            taps_s[9 * R:10 * R, :] = e1 + e2 + addv_r[:, :1]
            taps_s[4 * R:5 * R, :] = y5_r[...].reshape(R, HW)

        rc = lax.broadcasted_iota(jnp.int32, (R, 1), 0) % C
        for jj in range(NT):
            t = j * NT + jj
            y = taps_s[4 * R:5 * R, :]
            build_taps(y, store_center=False)
            eps0 = jnp.dot(a0_r[...], taps_s[...],
                           preferred_element_type=jnp.float32)

            c1 = coef_s[0, t]
            c2 = coef_s[1, t]
            sv = coef_s[2, t]
            tv = jnp.where(rc == 0, ct_s[0, t],
                           jnp.where(rc == 1, ct_s[1, t], ct_s[2, t]))
            nz = nz_r[:, jj].reshape(R, HW)
            y_new = c1 * y - c2 * eps0 - tv + sv * nz

            if jj == NT - 1:
                @pl.when(j == NJ - 1)
                def _finish():
                    out_r[...] = jnp.clip(y_new, -1.0, 1.0).reshape(
                        Bh, C, H, W)

            taps_s[4 * R:5 * R, :] = y_new

    return body


def _kron_taps(w_ock, Bh):
    C_out, _, C_in = w_ock.shape
    eyeB = jnp.eye(Bh, dtype=jnp.float32)
    big = (eyeB[:, None, None, :, None]
           * w_ock[None, :, :, None, :])
    return big.reshape(Bh * C_out, 9 * Bh * C_in)


def kernel(coef, temb_rev_p, w_lldc, w9p, bias, bw,
           lowlight, data_concate, brightness, y_init, noise_rev):
    B, C, H, W = lowlight.shape
    T = noise_rev.shape[1]
    HW = H * W
    Cp = w9p.shape[0]
    NC = 2
    Bh = B // NC
    R = Bh * C
    KP = 10 * R
    NT = 4

    w_y = w9p.reshape(Cp, 9, Cp)[:C, :, :C]
    A0 = jnp.concatenate(
        [_kron_taps(w_y, Bh), jnp.eye(R, dtype=jnp.float32)], axis=1)
    w_c = jnp.transpose(w_lldc, (3, 0, 1, 2)).reshape(C, 9, 2 * C)
    A_ll = _kron_taps(w_c[:, :, :C], Bh)
    A_dc = _kron_taps(w_c[:, :, C:], Bh)
    ct = (coef[1][:, None] * temb_rev_p[:, :C, 0]).T
    addv = (bias[None, :] + brightness.astype(jnp.float32)[:, None]
            * bw[None, :]).reshape(NC, R, 1)
    addv = addv + jnp.zeros((NC, R, 128), jnp.float32)

    out = pl.pallas_call(
        _make_body(H, W, Bh, R, KP, T, C, NT),
        out_shape=jax.ShapeDtypeStruct((B, C, H, W), jnp.float32),
        grid=(NC, T // NT),
        in_specs=[
            pl.BlockSpec(memory_space=pltpu.MemorySpace.SMEM),
            pl.BlockSpec(memory_space=pltpu.MemorySpace.SMEM),
            pl.BlockSpec((None, R, 128), lambda c, t: (c, 0, 0)),
            pl.BlockSpec((Bh, C, H, W), lambda c, t: (c, 0, 0, 0)),
            pl.BlockSpec((Bh, C, H, W), lambda c, t: (c, 0, 0, 0)),
            pl.BlockSpec((Bh, C, H, W), lambda c, t: (c, 0, 0, 0)),
            pl.BlockSpec((Bh, NT, C, H, W),
                         lambda c, t: (c, t, 0, 0, 0)),
            pl.BlockSpec((R, KP), lambda c, t: (0, 0)),
            pl.BlockSpec((R, 9 * R), lambda c, t: (0, 0)),
            pl.BlockSpec((R, 9 * R), lambda c, t: (0, 0)),
        ],
        out_specs=pl.BlockSpec((Bh, C, H, W), lambda c, t: (c, 0, 0, 0)),
        scratch_shapes=[
            pltpu.VMEM((KP, HW), jnp.float32),
            pltpu.VMEM((9, HW), jnp.float32),
        ],
        compiler_params=pltpu.CompilerParams(
            dimension_semantics=("parallel", "arbitrary")),
    )(coef, ct, addv, lowlight.astype(jnp.float32),
      data_concate.astype(jnp.float32), y_init.astype(jnp.float32),
      noise_rev.astype(jnp.float32), A0, A_ll, A_dc)

    return out


# speedup vs baseline: 5.2386x; 1.0150x over previous
import jax
import jax.numpy as jnp
from jax import lax
from jax.experimental import pallas as pl
from jax.experimental.pallas import tpu as pltpu


def _make_body(H, W, Bh, R, KP, T, C, NT):
    HW = H * W
    NJ = T // NT

    taps_meta = []
    for kh in range(3):
        for kw in range(3):
            dh, dw = kh - 1, kw - 1
            delta = dh * W + dw
            taps_meta.append(((-delta) % HW, dh, dw))

    def body(coef_s, ct_s, addv_r, ll_r, dc_r, y5_r, nz_r,
             a0_r, all_r, adc_r, out_r, taps_s, mask_s):
        j = pl.program_id(1)

        def build_taps(src, store_center):
            for k, (shift, dh, dw) in enumerate(taps_meta):
                if dh == 0 and dw == 0:
                    if store_center:
                        taps_s[k * R:(k + 1) * R, :] = src
                    continue
                tap = pltpu.roll(src, shift=shift, axis=1)
                tap = tap * mask_s[k:k + 1, :]
                taps_s[k * R:(k + 1) * R, :] = tap

        @pl.when(j == 0)
        def _init():
            hw_idx = lax.broadcasted_iota(jnp.int32, (1, HW), 1)
            h_pos = hw_idx // W
            w_pos = hw_idx % W
            for k, (_, dh, dw) in enumerate(taps_meta):
                valid = jnp.ones((1, HW), jnp.bool_)
                if dh == -1:
                    valid = jnp.logical_and(valid, h_pos >= 1)
                elif dh == 1:
                    valid = jnp.logical_and(valid, h_pos <= H - 2)
                if dw == -1:
                    valid = jnp.logical_and(valid, w_pos >= 1)
                elif dw == 1:
                    valid = jnp.logical_and(valid, w_pos <= W - 2)
                mask_s[k:k + 1, :] = valid.astype(jnp.float32)

            build_taps(ll_r[...].reshape(R, HW), store_center=True)
            e1 = jnp.dot(all_r[...], taps_s[0:9 * R, :],
                         preferred_element_type=jnp.float32)
            build_taps(dc_r[...].reshape(R, HW), store_center=True)
            e2 = jnp.dot(adc_r[...], taps_s[0:9 * R, :],
                         preferred_element_type=jnp.float32)
            taps_s[9 * R:10 * R, :] = e1 + e2 + addv_r[:, :1]
            taps_s[4 * R:5 * R, :] = y5_r[...].reshape(R, HW)

        rc = lax.broadcasted_iota(jnp.int32, (R, 1), 0) % C
        for jj in range(NT):
            t = j * NT + jj
            y = taps_s[4 * R:5 * R, :]
            build_taps(y, store_center=False)
            eps0 = jnp.dot(a0_r[...], taps_s[...],
                           preferred_element_type=jnp.float32)

            c1 = coef_s[0, t]
            c2 = coef_s[1, t]
            sv = coef_s[2, t]
            tv = jnp.where(rc == 0, ct_s[0, t],
                           jnp.where(rc == 1, ct_s[1, t], ct_s[2, t]))
            nz = nz_r[:, jj].reshape(R, HW)
            y_new = c1 * y - c2 * eps0 - tv + sv * nz

            if jj == NT - 1:
                @pl.when(j == NJ - 1)
                def _finish():
                    out_r[...] = jnp.clip(y_new, -1.0, 1.0).reshape(
                        Bh, C, H, W)

            taps_s[4 * R:5 * R, :] = y_new

    return body


def _kron_taps(w_ock, Bh):
    C_out, _, C_in = w_ock.shape
    eyeB = jnp.eye(Bh, dtype=jnp.float32)
    big = (eyeB[:, None, None, :, None]
           * w_ock[None, :, :, None, :])
    return big.reshape(Bh * C_out, 9 * Bh * C_in)


def kernel(coef, temb_rev_p, w_lldc, w9p, bias, bw,
           lowlight, data_concate, brightness, y_init, noise_rev):
    B, C, H, W = lowlight.shape
    T = noise_rev.shape[1]
    HW = H * W
    Cp = w9p.shape[0]
    NC = 2
    Bh = B // NC
    R = Bh * C
    KP = 10 * R
    NT = 8

    w_y = w9p.reshape(Cp, 9, Cp)[:C, :, :C]
    A0 = jnp.concatenate(
        [_kron_taps(w_y, Bh), jnp.eye(R, dtype=jnp.float32)], axis=1)
    w_c = jnp.transpose(w_lldc, (3, 0, 1, 2)).reshape(C, 9, 2 * C)
    A_ll = _kron_taps(w_c[:, :, :C], Bh)
    A_dc = _kron_taps(w_c[:, :, C:], Bh)
    ct = (coef[1][:, None] * temb_rev_p[:, :C, 0]).T
    addv = (bias[None, :] + brightness.astype(jnp.float32)[:, None]
            * bw[None, :]).reshape(NC, R, 1)
    addv = addv + jnp.zeros((NC, R, 128), jnp.float32)

    out = pl.pallas_call(
        _make_body(H, W, Bh, R, KP, T, C, NT),
        out_shape=jax.ShapeDtypeStruct((B, C, H, W), jnp.float32),
        grid=(NC, T // NT),
        in_specs=[
            pl.BlockSpec(memory_space=pltpu.MemorySpace.SMEM),
            pl.BlockSpec(memory_space=pltpu.MemorySpace.SMEM),
            pl.BlockSpec((None, R, 128), lambda c, t: (c, 0, 0)),
            pl.BlockSpec((Bh, C, H, W), lambda c, t: (c, 0, 0, 0)),
            pl.BlockSpec((Bh, C, H, W), lambda c, t: (c, 0, 0, 0)),
            pl.BlockSpec((Bh, C, H, W), lambda c, t: (c, 0, 0, 0)),
            pl.BlockSpec((Bh, NT, C, H, W),
                         lambda c, t: (c, t, 0, 0, 0)),
            pl.BlockSpec((R, KP), lambda c, t: (0, 0)),
            pl.BlockSpec((R, 9 * R), lambda c, t: (0, 0)),
            pl.BlockSpec((R, 9 * R), lambda c, t: (0, 0)),
        ],
        out_specs=pl.BlockSpec((Bh, C, H, W), lambda c, t: (c, 0, 0, 0)),
        scratch_shapes=[
            pltpu.VMEM((KP, HW), jnp.float32),
            pltpu.VMEM((9, HW), jnp.float32),
        ],
        compiler_params=pltpu.CompilerParams(
            dimension_semantics=("parallel", "arbitrary")),
    )(coef, ct, addv, lowlight.astype(jnp.float32),
      data_concate.astype(jnp.float32), y_init.astype(jnp.float32),
      noise_rev.astype(jnp.float32), A0, A_ll, A_dc)

    return out
